# Initial kernel scaffold; baseline (speedup 1.0000x reference)
import functools

import jax
import jax.numpy as jnp
from jax import lax
from jax.experimental import pallas as pl
from jax.experimental.pallas import tpu as pltpu
from jax.experimental.pallas import tpu_sc as plsc

N = 10000
E = 320000
D = 128
B = 16

NUM_TILES = 16
EDGES_PER_TILE = E // NUM_TILES
CHUNK = 80
NCHUNK = EDGES_PER_TILE // CHUNK
ROWS_PER_TILE = N // NUM_TILES
DEG_PAD = 10240

_HIGH = lax.Precision.HIGHEST


def _sc_mesh():
    return plsc.VectorSubcoreMesh(core_axis_name="c", subcore_axis_name="s")


@functools.partial(
    pl.kernel,
    out_type=jax.ShapeDtypeStruct((2, DEG_PAD), jnp.float32),
    mesh=_sc_mesh(),
    scratch_types=[
        pltpu.VMEM_SHARED((DEG_PAD,), jnp.float32),
        pltpu.VMEM((CHUNK,), jnp.int32),
        pltpu.VMEM((CHUNK,), jnp.float32),
        pltpu.VMEM((640,), jnp.float32),
    ],
)
def _sc_degree(dst_hbm, out_hbm, acc, idx_v, ones_v, zbuf):
    c = lax.axis_index("c")
    s = lax.axis_index("s")
    for j in range(640 // 16):
        zbuf[pl.ds(j * 16, 16)] = jnp.zeros((16,), jnp.float32)
    for j in range(CHUNK // 16):
        ones_v[pl.ds(j * 16, 16)] = jnp.ones((16,), jnp.float32)
    pltpu.sync_copy(zbuf, acc.at[pl.ds(s * 640, 640)])
    plsc.subcore_barrier()

    def body(i, carry):
        base = pl.multiple_of(s * EDGES_PER_TILE + i * CHUNK, 8)
        pltpu.sync_copy(dst_hbm.at[c, pl.ds(base, CHUNK)], idx_v)
        pltpu.sync_copy(ones_v, acc.at[idx_v], add=True)
        return carry

    lax.fori_loop(0, NCHUNK, body, 0)
    plsc.subcore_barrier()
    pltpu.sync_copy(acc.at[pl.ds(s * 640, 640)], out_hbm.at[c, pl.ds(s * 640, 640)])


@functools.partial(
    pl.kernel,
    out_type=jax.ShapeDtypeStruct((2, N, D), jnp.float32),
    mesh=_sc_mesh(),
    scratch_types=[
        pltpu.VMEM_SHARED((N, D), jnp.float32),
        pltpu.VMEM((CHUNK,), jnp.int32),
        pltpu.VMEM((CHUNK,), jnp.int32),
        pltpu.VMEM((CHUNK, D), jnp.float32),
        pltpu.SemaphoreType.DMA,
    ],
)
def _sc_edge_agg(y_hbm, src_hbm, dst_hbm, zeros_hbm, out_hbm,
                 acc, src_v, dst_v, rows_v, sem):
    c = lax.axis_index("c")
    s = lax.axis_index("s")
    pltpu.sync_copy(zeros_hbm.at[pl.ds(s * ROWS_PER_TILE, ROWS_PER_TILE)],
                    acc.at[pl.ds(s * ROWS_PER_TILE, ROWS_PER_TILE)])
    plsc.subcore_barrier()

    def body(i, carry):
        base = pl.multiple_of(s * EDGES_PER_TILE + i * CHUNK, 8)
        pltpu.sync_copy(src_hbm.at[c, pl.ds(base, CHUNK)], src_v)
        pltpu.async_copy(y_hbm.at[src_v], rows_v, sem).wait()
        pltpu.sync_copy(dst_hbm.at[c, pl.ds(base, CHUNK)], dst_v)
        pltpu.sync_copy(rows_v, acc.at[dst_v], add=True)
        return carry

    lax.fori_loop(0, NCHUNK, body, 0)
    plsc.subcore_barrier()
    pltpu.sync_copy(acc.at[pl.ds(s * ROWS_PER_TILE, ROWS_PER_TILE)],
                    out_hbm.at[c, pl.ds(s * ROWS_PER_TILE, ROWS_PER_TILE)])


ROW_BLK = 2500
NROW = N // ROW_BLK


def _pre_body(x_ref, w0t_ref, b0_ref, cw_ref, deg_ref, h0_ref, dinv_ref, y_ref):
    x = x_ref[0]
    out0 = jnp.maximum(
        jnp.dot(x, w0t_ref[0], precision=_HIGH) + b0_ref[0], 0.0)
    dinv = lax.rsqrt(1.0 + deg_ref[0])
    h0_ref[0] = out0
    dinv_ref[0] = dinv
    y_ref[0] = dinv * jnp.dot(out0, cw_ref[0], precision=_HIGH)


def _tc_pre(x, w0t, b0, cw, deg):
    eb = lambda e, r: (e, 0, 0)
    rb = lambda e, r: (e, r, 0)
    return pl.pallas_call(
        _pre_body,
        grid=(2, NROW),
        in_specs=[
            pl.BlockSpec((1, ROW_BLK, D), rb),
            pl.BlockSpec((1, D, D), eb),
            pl.BlockSpec((1, 1, D), eb),
            pl.BlockSpec((1, D, D), eb),
            pl.BlockSpec((1, ROW_BLK, 1), rb),
        ],
        out_specs=[
            pl.BlockSpec((1, ROW_BLK, D), rb),
            pl.BlockSpec((1, ROW_BLK, 1), rb),
            pl.BlockSpec((1, ROW_BLK, D), rb),
        ],
        out_shape=[
            jax.ShapeDtypeStruct((2, N, D), jnp.float32),
            jax.ShapeDtypeStruct((2, N, 1), jnp.float32),
            jax.ShapeDtypeStruct((2, N, D), jnp.float32),
        ],
    )(x, w0t, b0, cw, deg)


def _gru_body(want_y, y_ref, agg_ref, dinv_ref, cb_ref, h_ref,
              wiht_ref, whht_ref, bih_ref, bhh_ref, cw_ref, *out_refs):
    dinv = dinv_ref[0]
    m = jnp.maximum(dinv * (y_ref[0] + agg_ref[0]) + cb_ref[0], 0.0)
    gi = jnp.dot(m, wiht_ref[0], precision=_HIGH) + bih_ref[0]
    gh = jnp.dot(h_ref[0], whht_ref[0], precision=_HIGH) + bhh_ref[0]
    r = jax.nn.sigmoid(gi[:, :D] + gh[:, :D])
    z = jax.nn.sigmoid(gi[:, D:2 * D] + gh[:, D:2 * D])
    n = jnp.tanh(gi[:, 2 * D:] + r * gh[:, 2 * D:])
    h_new = (1.0 - z) * n + z * h_ref[0]
    out_refs[0][0] = h_new
    if want_y:
        out_refs[1][0] = dinv * jnp.dot(h_new, cw_ref[0], precision=_HIGH)


def _tc_gru(y, agg, dinv, cb, h, wiht, whht, bih, bhh, cw, want_y):
    eb = lambda e, r: (e, 0, 0)
    rb = lambda e, r: (e, r, 0)
    out_specs = [pl.BlockSpec((1, ROW_BLK, D), rb)]
    out_shape = [jax.ShapeDtypeStruct((2, N, D), jnp.float32)]
    if want_y:
        out_specs.append(pl.BlockSpec((1, ROW_BLK, D), rb))
        out_shape.append(jax.ShapeDtypeStruct((2, N, D), jnp.float32))
    return pl.pallas_call(
        functools.partial(_gru_body, want_y),
        grid=(2, NROW),
        in_specs=[
            pl.BlockSpec((1, ROW_BLK, D), rb),
            pl.BlockSpec((1, ROW_BLK, D), rb),
            pl.BlockSpec((1, ROW_BLK, 1), rb),
            pl.BlockSpec((1, 1, D), eb),
            pl.BlockSpec((1, ROW_BLK, D), rb),
            pl.BlockSpec((1, D, 3 * D), eb),
            pl.BlockSpec((1, D, 3 * D), eb),
            pl.BlockSpec((1, 1, 3 * D), eb),
            pl.BlockSpec((1, 1, 3 * D), eb),
            pl.BlockSpec((1, D, D), eb),
        ],
        out_specs=out_specs,
        out_shape=out_shape,
    )(y, agg, dinv, cb, h, wiht, whht, bih, bhh, cw)


def _final_body(h_ref, brow_ref, wiht_ref, whht_ref, bih_ref, bhh_ref,
                f1t_ref, f1b_ref, f2t_ref, f2b_ref, out_ref):
    q_stars = []
    for e in range(2):
        x = h_ref[e]
        brow = brow_ref[e]
        gid = lax.broadcasted_iota(jnp.int32, (B, N), 0)
        maskT = brow == gid
        hl = jnp.zeros((B, D), jnp.float32)
        cl = jnp.zeros((B, D), jnp.float32)
        q_star = jnp.zeros((B, 2 * D), jnp.float32)
        for _ in range(3):
            gates = (jnp.dot(q_star, wiht_ref[e], precision=_HIGH) + bih_ref[e]
                     + jnp.dot(hl, whht_ref[e], precision=_HIGH) + bhh_ref[e])
            ii = jax.nn.sigmoid(gates[:, :D])
            ff = jax.nn.sigmoid(gates[:, D:2 * D])
            gg = jnp.tanh(gates[:, 2 * D:3 * D])
            oo = jax.nn.sigmoid(gates[:, 3 * D:])
            cl = ff * cl + ii * gg
            hl = oo * jnp.tanh(cl)
            q = hl
            st = lax.dot_general(q, x, (((1,), (1,)), ((), ())),
                                 precision=_HIGH)
            smt = jnp.where(maskT, st, -jnp.inf)
            emax = jnp.max(smt, axis=1, keepdims=True)
            emax = jnp.where(jnp.isfinite(emax), emax, 0.0)
            pt = jnp.exp(smt - emax)
            denom = jnp.sum(pt, axis=1, keepdims=True)
            at = pt / (denom + 1e-16)
            r = jnp.dot(at, x, precision=_HIGH)
            q_star = jnp.concatenate([q, r], axis=1)
        q_stars.append(q_star)
    cat = jnp.concatenate(q_stars, axis=1)
    hfc = jnp.maximum(jnp.dot(cat, f1t_ref[...], precision=_HIGH)
                      + f1b_ref[...], 0.0)
    out_ref[...] = jnp.dot(hfc, f2t_ref[...], precision=_HIGH) + f2b_ref[...]


def _tc_final(h, brow, wiht, whht, bih, bhh, f1t, f1b, f2t, f2b):
    return pl.pallas_call(
        _final_body,
        out_shape=jax.ShapeDtypeStruct((B, 1), jnp.float32),
    )(h, brow, wiht, whht, bih, bhh, f1t, f1b, f2t, f2b)


def kernel(x1, x2, edge_index1, edge_index2, x1_batch, x2_batch,
           e1_lin0_W, e1_lin0_b, e1_conv_W, e1_conv_b,
           e1_gru_Wih, e1_gru_Whh, e1_gru_bih, e1_gru_bhh,
           e1_lstm_Wih, e1_lstm_Whh, e1_lstm_bih, e1_lstm_bhh,
           e2_lin0_W, e2_lin0_b, e2_conv_W, e2_conv_b,
           e2_gru_Wih, e2_gru_Whh, e2_gru_bih, e2_gru_bhh,
           e2_lstm_Wih, e2_lstm_Whh, e2_lstm_bih, e2_lstm_bhh,
           fc1_W, fc1_b, fc2_W, fc2_b):
    x = jnp.stack([x1, x2])
    src = jnp.stack([edge_index1[0], edge_index2[0] + N])
    dst = jnp.stack([edge_index1[1], edge_index2[1]])
    brow = jnp.stack([x1_batch, x2_batch])[:, None, :]

    w0t = jnp.stack([e1_lin0_W.T, e2_lin0_W.T])
    b0 = jnp.stack([e1_lin0_b, e2_lin0_b])[:, None, :]
    cw = jnp.stack([e1_conv_W, e2_conv_W])
    cb = jnp.stack([e1_conv_b, e2_conv_b])[:, None, :]
    wiht = jnp.stack([e1_gru_Wih.T, e2_gru_Wih.T])
    whht = jnp.stack([e1_gru_Whh.T, e2_gru_Whh.T])
    bih = jnp.stack([e1_gru_bih, e2_gru_bih])[:, None, :]
    bhh = jnp.stack([e1_gru_bhh, e2_gru_bhh])[:, None, :]
    lwiht = jnp.stack([e1_lstm_Wih.T, e2_lstm_Wih.T])
    lwhht = jnp.stack([e1_lstm_Whh.T, e2_lstm_Whh.T])
    lbih = jnp.stack([e1_lstm_bih, e2_lstm_bih])[:, None, :]
    lbhh = jnp.stack([e1_lstm_bhh, e2_lstm_bhh])[:, None, :]

    zeros_mat = jnp.zeros((N, D), jnp.float32)

    deg = _sc_degree(dst)
    deg = deg[:, :N, None]

    h, dinv, y = _tc_pre(x, w0t, b0, cw, deg)
    for layer in range(3):
        agg = _sc_edge_agg(y.reshape(2 * N, D), src, dst, zeros_mat)
        want_y = layer < 2
        outs = _tc_gru(y, agg, dinv, cb, h, wiht, whht, bih, bhh, cw, want_y)
        if want_y:
            h, y = outs
        else:
            h = outs[0]

    out = _tc_final(h, brow, lwiht, lwhht, lbih, lbhh,
                    fc1_W.T, fc1_b[None, :], fc2_W.T, fc2_b[None, :])
    return out.reshape(-1)


# speedup vs baseline: 8.3419x; 8.3419x over previous
import functools

import jax
import jax.numpy as jnp
from jax import lax
from jax.experimental import pallas as pl
from jax.experimental.pallas import tpu as pltpu
from jax.experimental.pallas import tpu_sc as plsc

N = 10000
E = 320000
D = 128
B = 16

NUM_TILES = 16
EDGES_PER_TILE = E // NUM_TILES
CHUNK = 80
NCHUNK = EDGES_PER_TILE // CHUNK
NPAD = 10240
ROWS_PER_TILE = NPAD // NUM_TILES
DEG_PAD = 10240

_HIGH = lax.Precision.HIGHEST


def _sc_mesh():
    return plsc.VectorSubcoreMesh(core_axis_name="c", subcore_axis_name="s")


@functools.partial(
    pl.kernel,
    out_type=jax.ShapeDtypeStruct((2 * DEG_PAD,), jnp.float32),
    mesh=_sc_mesh(),
    scratch_types=[
        pltpu.VMEM_SHARED((DEG_PAD,), jnp.float32),
        pltpu.VMEM((CHUNK,), jnp.int32),
        pltpu.VMEM((CHUNK,), jnp.float32),
        pltpu.VMEM((640,), jnp.float32),
    ],
)
def _sc_degree(dst_hbm, out_hbm, acc, idx_v, ones_v, zbuf):
    c = lax.axis_index("c")
    s = lax.axis_index("s")
    for j in range(640 // 16):
        zbuf[pl.ds(j * 16, 16)] = jnp.zeros((16,), jnp.float32)
    for j in range(CHUNK // 16):
        ones_v[pl.ds(j * 16, 16)] = jnp.ones((16,), jnp.float32)
    pltpu.sync_copy(zbuf, acc.at[pl.ds(s * 640, 640)])
    plsc.subcore_barrier()

    def body(i, carry):
        base = pl.multiple_of(c * E + s * EDGES_PER_TILE + i * CHUNK, 8)
        pltpu.sync_copy(dst_hbm.at[pl.ds(base, CHUNK)], idx_v)
        pltpu.sync_copy(ones_v, acc.at[idx_v], add=True)
        return carry

    lax.fori_loop(0, NCHUNK, body, 0)
    plsc.subcore_barrier()
    obase = pl.multiple_of(c * DEG_PAD + s * 640, 8)
    pltpu.sync_copy(acc.at[pl.ds(s * 640, 640)], out_hbm.at[pl.ds(obase, 640)])


@functools.partial(
    pl.kernel,
    out_type=jax.ShapeDtypeStruct((2, NPAD, D), jnp.float32),
    mesh=_sc_mesh(),
    scratch_types=[
        pltpu.VMEM_SHARED((NPAD, D), jnp.float32),
        pltpu.VMEM((CHUNK,), jnp.int32),
        pltpu.VMEM((CHUNK,), jnp.int32),
        pltpu.VMEM((CHUNK, D), jnp.float32),
        pltpu.SemaphoreType.DMA,
    ],
)
def _sc_edge_agg(y_hbm, src_hbm, dst_hbm, zeros_hbm, out_hbm,
                 acc, src_v, dst_v, rows_v, sem):
    c = lax.axis_index("c")
    s = lax.axis_index("s")
    pltpu.sync_copy(zeros_hbm.at[pl.ds(s * ROWS_PER_TILE, ROWS_PER_TILE)],
                    acc.at[pl.ds(s * ROWS_PER_TILE, ROWS_PER_TILE)])
    plsc.subcore_barrier()

    def body(i, carry):
        base = pl.multiple_of(c * E + s * EDGES_PER_TILE + i * CHUNK, 8)
        pltpu.sync_copy(src_hbm.at[pl.ds(base, CHUNK)], src_v)
        pltpu.async_copy(y_hbm.at[src_v], rows_v, sem).wait()
        pltpu.sync_copy(dst_hbm.at[pl.ds(base, CHUNK)], dst_v)
        pltpu.sync_copy(rows_v, acc.at[dst_v], add=True)
        return carry

    lax.fori_loop(0, NCHUNK, body, 0)
    plsc.subcore_barrier()
    pltpu.sync_copy(acc.at[pl.ds(s * ROWS_PER_TILE, ROWS_PER_TILE)],
                    out_hbm.at[c, pl.ds(s * ROWS_PER_TILE, ROWS_PER_TILE)])


ROW_BLK = 2000
NROW = N // ROW_BLK


def _pre_body(x_ref, w0t_ref, b0_ref, cw_ref, deg_ref, h0_ref, dinv_ref, y_ref):
    x = x_ref[0]
    out0 = jnp.maximum(
        jnp.dot(x, w0t_ref[0], precision=_HIGH) + b0_ref[0], 0.0)
    dinv = lax.rsqrt(1.0 + deg_ref[0])
    h0_ref[0] = out0
    dinv_ref[0] = dinv
    y_ref[0] = dinv * jnp.dot(out0, cw_ref[0], precision=_HIGH)


def _tc_pre(x, w0t, b0, cw, deg):
    eb = lambda e, r: (e, 0, 0)
    rb = lambda e, r: (e, r, 0)
    return pl.pallas_call(
        _pre_body,
        grid=(2, NROW),
        in_specs=[
            pl.BlockSpec((1, ROW_BLK, D), rb),
            pl.BlockSpec((1, D, D), eb),
            pl.BlockSpec((1, 1, D), eb),
            pl.BlockSpec((1, D, D), eb),
            pl.BlockSpec((1, ROW_BLK, 1), rb),
        ],
        out_specs=[
            pl.BlockSpec((1, ROW_BLK, D), rb),
            pl.BlockSpec((1, ROW_BLK, 1), rb),
            pl.BlockSpec((1, ROW_BLK, D), rb),
        ],
        out_shape=[
            jax.ShapeDtypeStruct((2, N, D), jnp.float32),
            jax.ShapeDtypeStruct((2, N, 1), jnp.float32),
            jax.ShapeDtypeStruct((2, N, D), jnp.float32),
        ],
    )(x, w0t, b0, cw, deg)


def _gru_body(want_y, y_ref, agg_ref, dinv_ref, cb_ref, h_ref,
              wiht_ref, whht_ref, bih_ref, bhh_ref, cw_ref, *out_refs):
    dinv = dinv_ref[0]
    m = jnp.maximum(dinv * (y_ref[0] + agg_ref[0]) + cb_ref[0], 0.0)
    gi = jnp.dot(m, wiht_ref[0], precision=_HIGH) + bih_ref[0]
    gh = jnp.dot(h_ref[0], whht_ref[0], precision=_HIGH) + bhh_ref[0]
    r = jax.nn.sigmoid(gi[:, :D] + gh[:, :D])
    z = jax.nn.sigmoid(gi[:, D:2 * D] + gh[:, D:2 * D])
    n = jnp.tanh(gi[:, 2 * D:] + r * gh[:, 2 * D:])
    h_new = (1.0 - z) * n + z * h_ref[0]
    out_refs[0][0] = h_new
    if want_y:
        out_refs[1][0] = dinv * jnp.dot(h_new, cw_ref[0], precision=_HIGH)


def _tc_gru(y, agg, dinv, cb, h, wiht, whht, bih, bhh, cw, want_y):
    eb = lambda e, r: (e, 0, 0)
    rb = lambda e, r: (e, r, 0)
    out_specs = [pl.BlockSpec((1, ROW_BLK, D), rb)]
    out_shape = [jax.ShapeDtypeStruct((2, N, D), jnp.float32)]
    if want_y:
        out_specs.append(pl.BlockSpec((1, ROW_BLK, D), rb))
        out_shape.append(jax.ShapeDtypeStruct((2, N, D), jnp.float32))
    return pl.pallas_call(
        functools.partial(_gru_body, want_y),
        grid=(2, NROW),
        in_specs=[
            pl.BlockSpec((1, ROW_BLK, D), rb),
            pl.BlockSpec((1, ROW_BLK, D), rb),
            pl.BlockSpec((1, ROW_BLK, 1), rb),
            pl.BlockSpec((1, 1, D), eb),
            pl.BlockSpec((1, ROW_BLK, D), rb),
            pl.BlockSpec((1, D, 3 * D), eb),
            pl.BlockSpec((1, D, 3 * D), eb),
            pl.BlockSpec((1, 1, 3 * D), eb),
            pl.BlockSpec((1, 1, 3 * D), eb),
            pl.BlockSpec((1, D, D), eb),
        ],
        out_specs=out_specs,
        out_shape=out_shape,
    )(y, agg, dinv, cb, h, wiht, whht, bih, bhh, cw)


def _final_body(h_ref, brow_ref, wiht_ref, whht_ref, bih_ref, bhh_ref,
                f1t_ref, f1b_ref, f2t_ref, f2b_ref, out_ref):
    q_stars = []
    for e in range(2):
        x = h_ref[e]
        brow = brow_ref[e]
        gid = lax.broadcasted_iota(jnp.int32, (B, N), 0)
        maskT = brow == gid
        hl = jnp.zeros((B, D), jnp.float32)
        cl = jnp.zeros((B, D), jnp.float32)
        q_star = jnp.zeros((B, 2 * D), jnp.float32)
        for _ in range(3):
            gates = (jnp.dot(q_star, wiht_ref[e], precision=_HIGH) + bih_ref[e]
                     + jnp.dot(hl, whht_ref[e], precision=_HIGH) + bhh_ref[e])
            ii = jax.nn.sigmoid(gates[:, :D])
            ff = jax.nn.sigmoid(gates[:, D:2 * D])
            gg = jnp.tanh(gates[:, 2 * D:3 * D])
            oo = jax.nn.sigmoid(gates[:, 3 * D:])
            cl = ff * cl + ii * gg
            hl = oo * jnp.tanh(cl)
            q = hl
            st = lax.dot_general(q, x, (((1,), (1,)), ((), ())),
                                 precision=_HIGH)
            smt = jnp.where(maskT, st, -jnp.inf)
            emax = jnp.max(smt, axis=1, keepdims=True)
            emax = jnp.where(jnp.isfinite(emax), emax, 0.0)
            pt = jnp.exp(smt - emax)
            denom = jnp.sum(pt, axis=1, keepdims=True)
            at = pt / (denom + 1e-16)
            r = jnp.dot(at, x, precision=_HIGH)
            q_star = jnp.concatenate([q, r], axis=1)
        q_stars.append(q_star)
    cat = jnp.concatenate(q_stars, axis=1)
    hfc = jnp.maximum(jnp.dot(cat, f1t_ref[...], precision=_HIGH)
                      + f1b_ref[...], 0.0)
    out_ref[...] = jnp.dot(hfc, f2t_ref[...], precision=_HIGH) + f2b_ref[...]


def _tc_final(h, brow, wiht, whht, bih, bhh, f1t, f1b, f2t, f2b):
    return pl.pallas_call(
        _final_body,
        out_shape=jax.ShapeDtypeStruct((B, 1), jnp.float32),
    )(h, brow, wiht, whht, bih, bhh, f1t, f1b, f2t, f2b)


def kernel(x1, x2, edge_index1, edge_index2, x1_batch, x2_batch,
           e1_lin0_W, e1_lin0_b, e1_conv_W, e1_conv_b,
           e1_gru_Wih, e1_gru_Whh, e1_gru_bih, e1_gru_bhh,
           e1_lstm_Wih, e1_lstm_Whh, e1_lstm_bih, e1_lstm_bhh,
           e2_lin0_W, e2_lin0_b, e2_conv_W, e2_conv_b,
           e2_gru_Wih, e2_gru_Whh, e2_gru_bih, e2_gru_bhh,
           e2_lstm_Wih, e2_lstm_Whh, e2_lstm_bih, e2_lstm_bhh,
           fc1_W, fc1_b, fc2_W, fc2_b):
    x = jnp.stack([x1, x2])
    src = jnp.concatenate([edge_index1[0], edge_index2[0] + N])
    dst = jnp.concatenate([edge_index1[1], edge_index2[1]])
    brow = jnp.stack([x1_batch, x2_batch])[:, None, :]

    w0t = jnp.stack([e1_lin0_W.T, e2_lin0_W.T])
    b0 = jnp.stack([e1_lin0_b, e2_lin0_b])[:, None, :]
    cw = jnp.stack([e1_conv_W, e2_conv_W])
    cb = jnp.stack([e1_conv_b, e2_conv_b])[:, None, :]
    wiht = jnp.stack([e1_gru_Wih.T, e2_gru_Wih.T])
    whht = jnp.stack([e1_gru_Whh.T, e2_gru_Whh.T])
    bih = jnp.stack([e1_gru_bih, e2_gru_bih])[:, None, :]
    bhh = jnp.stack([e1_gru_bhh, e2_gru_bhh])[:, None, :]
    lwiht = jnp.stack([e1_lstm_Wih.T, e2_lstm_Wih.T])
    lwhht = jnp.stack([e1_lstm_Whh.T, e2_lstm_Whh.T])
    lbih = jnp.stack([e1_lstm_bih, e2_lstm_bih])[:, None, :]
    lbhh = jnp.stack([e1_lstm_bhh, e2_lstm_bhh])[:, None, :]

    zeros_mat = jnp.zeros((NPAD, D), jnp.float32)

    deg = _sc_degree(dst)
    deg = deg.reshape(2, DEG_PAD)[:, :N, None]

    h, dinv, y = _tc_pre(x, w0t, b0, cw, deg)
    for layer in range(3):
        agg = _sc_edge_agg(y.reshape(2 * N, D), src, dst, zeros_mat)
        want_y = layer < 2
        outs = _tc_gru(y, agg, dinv, cb, h, wiht, whht, bih, bhh, cw, want_y)
        if want_y:
            h, y = outs
        else:
            h = outs[0]

    out = _tc_final(h, brow, lwiht, lwhht, lbih, lbhh,
                    fc1_W.T, fc1_b[None, :], fc2_W.T, fc2_b[None, :])
    return out.reshape(-1)


# speedup vs baseline: 17.0020x; 2.0381x over previous
import functools

import jax
import jax.numpy as jnp
from jax import lax
from jax.experimental import pallas as pl
from jax.experimental.pallas import tpu as pltpu
from jax.experimental.pallas import tpu_sc as plsc

N = 10000
E = 320000
D = 128
B = 16

NUM_TILES = 16
EDGES_PER_TILE = E // NUM_TILES
CHUNK = 80
NCHUNK = EDGES_PER_TILE // CHUNK
NPAD = 10240
ROWS_PER_TILE = NPAD // NUM_TILES
DEG_PAD = 10240

_HIGH = lax.Precision.HIGHEST


def _sc_mesh():
    return plsc.VectorSubcoreMesh(core_axis_name="c", subcore_axis_name="s")


@functools.partial(
    pl.kernel,
    out_type=jax.ShapeDtypeStruct((2 * DEG_PAD,), jnp.float32),
    mesh=_sc_mesh(),
    scratch_types=[
        pltpu.VMEM_SHARED((DEG_PAD,), jnp.float32),
        pltpu.VMEM((CHUNK,), jnp.int32),
        pltpu.VMEM((CHUNK,), jnp.float32),
        pltpu.VMEM((640,), jnp.float32),
    ],
)
def _sc_degree(dst_hbm, out_hbm, acc, idx_v, ones_v, zbuf):
    c = lax.axis_index("c")
    s = lax.axis_index("s")
    for j in range(640 // 16):
        zbuf[pl.ds(j * 16, 16)] = jnp.zeros((16,), jnp.float32)
    for j in range(CHUNK // 16):
        ones_v[pl.ds(j * 16, 16)] = jnp.ones((16,), jnp.float32)
    pltpu.sync_copy(zbuf, acc.at[pl.ds(s * 640, 640)])
    plsc.subcore_barrier()

    def body(i, carry):
        base = pl.multiple_of(c * E + s * EDGES_PER_TILE + i * CHUNK, 8)
        pltpu.sync_copy(dst_hbm.at[pl.ds(base, CHUNK)], idx_v)
        pltpu.sync_copy(ones_v, acc.at[idx_v], add=True)
        return carry

    lax.fori_loop(0, NCHUNK, body, 0)
    plsc.subcore_barrier()
    obase = pl.multiple_of(c * DEG_PAD + s * 640, 8)
    pltpu.sync_copy(acc.at[pl.ds(s * 640, 640)], out_hbm.at[pl.ds(obase, 640)])


ECHUNK = 100
EROWS = EDGES_PER_TILE // ECHUNK
ROWS_PER_SC = E // ECHUNK
SB = 40


@functools.partial(
    pl.kernel,
    out_type=jax.ShapeDtypeStruct((2, NPAD, D), jnp.float32),
    mesh=_sc_mesh(),
    scratch_types=[
        pltpu.VMEM_SHARED((NPAD, D), jnp.float32),
        pltpu.VMEM((SB, ECHUNK), jnp.int32),
        pltpu.VMEM((SB, ECHUNK), jnp.int32),
        pltpu.VMEM((ECHUNK, D), jnp.float32),
        pltpu.VMEM((ECHUNK, D), jnp.float32),
        pltpu.SemaphoreType.DMA,
        pltpu.SemaphoreType.DMA,
    ],
)
def _sc_edge_agg(y_hbm, src_hbm, dst_hbm, zeros_hbm, out_hbm,
                 acc, sidx, didx, rows_a, rows_b, sem_a, sem_b):
    c = lax.axis_index("c")
    s = lax.axis_index("s")
    pltpu.sync_copy(zeros_hbm.at[pl.ds(s * ROWS_PER_TILE, ROWS_PER_TILE)],
                    acc.at[pl.ds(s * ROWS_PER_TILE, ROWS_PER_TILE)])
    plsc.subcore_barrier()

    def superblock(ob, carry):
        rowbase = pl.multiple_of(
            c * ROWS_PER_SC + s * EROWS + ob * SB, 8)
        pltpu.sync_copy(src_hbm.at[pl.ds(rowbase, SB)], sidx)
        pltpu.sync_copy(dst_hbm.at[pl.ds(rowbase, SB)], didx)
        pltpu.async_copy(y_hbm.at[sidx.at[0]], rows_a, sem_a)

        def pair(p, carry2):
            j0 = 2 * p
            pltpu.async_copy(y_hbm.at[sidx.at[j0 + 1]], rows_b, sem_b)
            pltpu.make_async_copy(y_hbm.at[sidx.at[j0]], rows_a, sem_a).wait()
            pltpu.sync_copy(rows_a, acc.at[didx.at[j0]], add=True)

            @pl.when(p < SB // 2 - 1)
            def _():
                pltpu.async_copy(y_hbm.at[sidx.at[j0 + 2]], rows_a, sem_a)

            pltpu.make_async_copy(y_hbm.at[sidx.at[j0 + 1]], rows_b, sem_b).wait()
            pltpu.sync_copy(rows_b, acc.at[didx.at[j0 + 1]], add=True)
            return carry2

        lax.fori_loop(0, SB // 2, pair, 0)
        return carry

    lax.fori_loop(0, EROWS // SB, superblock, 0)
    plsc.subcore_barrier()
    pltpu.sync_copy(acc.at[pl.ds(s * ROWS_PER_TILE, ROWS_PER_TILE)],
                    out_hbm.at[c, pl.ds(s * ROWS_PER_TILE, ROWS_PER_TILE)])


ROW_BLK = 2000
NROW = N // ROW_BLK


def _pre_body(x_ref, w0t_ref, b0_ref, cw_ref, deg_ref, h0_ref, dinv_ref, y_ref):
    x = x_ref[0]
    out0 = jnp.maximum(
        jnp.dot(x, w0t_ref[0], precision=_HIGH) + b0_ref[0], 0.0)
    dinv = lax.rsqrt(1.0 + deg_ref[0])
    h0_ref[0] = out0
    dinv_ref[0] = dinv
    y_ref[0] = dinv * jnp.dot(out0, cw_ref[0], precision=_HIGH)


def _tc_pre(x, w0t, b0, cw, deg):
    eb = lambda e, r: (e, 0, 0)
    rb = lambda e, r: (e, r, 0)
    return pl.pallas_call(
        _pre_body,
        grid=(2, NROW),
        in_specs=[
            pl.BlockSpec((1, ROW_BLK, D), rb),
            pl.BlockSpec((1, D, D), eb),
            pl.BlockSpec((1, 1, D), eb),
            pl.BlockSpec((1, D, D), eb),
            pl.BlockSpec((1, ROW_BLK, 1), rb),
        ],
        out_specs=[
            pl.BlockSpec((1, ROW_BLK, D), rb),
            pl.BlockSpec((1, ROW_BLK, 1), rb),
            pl.BlockSpec((1, ROW_BLK, D), rb),
        ],
        out_shape=[
            jax.ShapeDtypeStruct((2, N, D), jnp.float32),
            jax.ShapeDtypeStruct((2, N, 1), jnp.float32),
            jax.ShapeDtypeStruct((2, N, D), jnp.float32),
        ],
    )(x, w0t, b0, cw, deg)


def _gru_body(want_y, y_ref, agg_ref, dinv_ref, cb_ref, h_ref,
              wiht_ref, whht_ref, bih_ref, bhh_ref, cw_ref, *out_refs):
    dinv = dinv_ref[0]
    m = jnp.maximum(dinv * (y_ref[0] + agg_ref[0]) + cb_ref[0], 0.0)
    gi = jnp.dot(m, wiht_ref[0], precision=_HIGH) + bih_ref[0]
    gh = jnp.dot(h_ref[0], whht_ref[0], precision=_HIGH) + bhh_ref[0]
    r = jax.nn.sigmoid(gi[:, :D] + gh[:, :D])
    z = jax.nn.sigmoid(gi[:, D:2 * D] + gh[:, D:2 * D])
    n = jnp.tanh(gi[:, 2 * D:] + r * gh[:, 2 * D:])
    h_new = (1.0 - z) * n + z * h_ref[0]
    out_refs[0][0] = h_new
    if want_y:
        out_refs[1][0] = dinv * jnp.dot(h_new, cw_ref[0], precision=_HIGH)


def _tc_gru(y, agg, dinv, cb, h, wiht, whht, bih, bhh, cw, want_y):
    eb = lambda e, r: (e, 0, 0)
    rb = lambda e, r: (e, r, 0)
    out_specs = [pl.BlockSpec((1, ROW_BLK, D), rb)]
    out_shape = [jax.ShapeDtypeStruct((2, N, D), jnp.float32)]
    if want_y:
        out_specs.append(pl.BlockSpec((1, ROW_BLK, D), rb))
        out_shape.append(jax.ShapeDtypeStruct((2, N, D), jnp.float32))
    return pl.pallas_call(
        functools.partial(_gru_body, want_y),
        grid=(2, NROW),
        in_specs=[
            pl.BlockSpec((1, ROW_BLK, D), rb),
            pl.BlockSpec((1, ROW_BLK, D), rb),
            pl.BlockSpec((1, ROW_BLK, 1), rb),
            pl.BlockSpec((1, 1, D), eb),
            pl.BlockSpec((1, ROW_BLK, D), rb),
            pl.BlockSpec((1, D, 3 * D), eb),
            pl.BlockSpec((1, D, 3 * D), eb),
            pl.BlockSpec((1, 1, 3 * D), eb),
            pl.BlockSpec((1, 1, 3 * D), eb),
            pl.BlockSpec((1, D, D), eb),
        ],
        out_specs=out_specs,
        out_shape=out_shape,
    )(y, agg, dinv, cb, h, wiht, whht, bih, bhh, cw)


def _final_body(h_ref, brow_ref, wiht_ref, whht_ref, bih_ref, bhh_ref,
                f1t_ref, f1b_ref, f2t_ref, f2b_ref, out_ref):
    q_stars = []
    for e in range(2):
        x = h_ref[e]
        brow = brow_ref[e]
        gid = lax.broadcasted_iota(jnp.int32, (B, N), 0)
        maskT = brow == gid
        hl = jnp.zeros((B, D), jnp.float32)
        cl = jnp.zeros((B, D), jnp.float32)
        q_star = jnp.zeros((B, 2 * D), jnp.float32)
        for _ in range(3):
            gates = (jnp.dot(q_star, wiht_ref[e], precision=_HIGH) + bih_ref[e]
                     + jnp.dot(hl, whht_ref[e], precision=_HIGH) + bhh_ref[e])
            ii = jax.nn.sigmoid(gates[:, :D])
            ff = jax.nn.sigmoid(gates[:, D:2 * D])
            gg = jnp.tanh(gates[:, 2 * D:3 * D])
            oo = jax.nn.sigmoid(gates[:, 3 * D:])
            cl = ff * cl + ii * gg
            hl = oo * jnp.tanh(cl)
            q = hl
            st = lax.dot_general(q, x, (((1,), (1,)), ((), ())),
                                 precision=_HIGH)
            smt = jnp.where(maskT, st, -jnp.inf)
            emax = jnp.max(smt, axis=1, keepdims=True)
            emax = jnp.where(jnp.isfinite(emax), emax, 0.0)
            pt = jnp.exp(smt - emax)
            denom = jnp.sum(pt, axis=1, keepdims=True)
            at = pt / (denom + 1e-16)
            r = jnp.dot(at, x, precision=_HIGH)
            q_star = jnp.concatenate([q, r], axis=1)
        q_stars.append(q_star)
    cat = jnp.concatenate(q_stars, axis=1)
    hfc = jnp.maximum(jnp.dot(cat, f1t_ref[...], precision=_HIGH)
                      + f1b_ref[...], 0.0)
    out_ref[...] = jnp.dot(hfc, f2t_ref[...], precision=_HIGH) + f2b_ref[...]


def _tc_final(h, brow, wiht, whht, bih, bhh, f1t, f1b, f2t, f2b):
    return pl.pallas_call(
        _final_body,
        out_shape=jax.ShapeDtypeStruct((B, 1), jnp.float32),
    )(h, brow, wiht, whht, bih, bhh, f1t, f1b, f2t, f2b)


def kernel(x1, x2, edge_index1, edge_index2, x1_batch, x2_batch,
           e1_lin0_W, e1_lin0_b, e1_conv_W, e1_conv_b,
           e1_gru_Wih, e1_gru_Whh, e1_gru_bih, e1_gru_bhh,
           e1_lstm_Wih, e1_lstm_Whh, e1_lstm_bih, e1_lstm_bhh,
           e2_lin0_W, e2_lin0_b, e2_conv_W, e2_conv_b,
           e2_gru_Wih, e2_gru_Whh, e2_gru_bih, e2_gru_bhh,
           e2_lstm_Wih, e2_lstm_Whh, e2_lstm_bih, e2_lstm_bhh,
           fc1_W, fc1_b, fc2_W, fc2_b):
    x = jnp.stack([x1, x2])
    src = jnp.concatenate([edge_index1[0], edge_index2[0] + N])
    dst = jnp.concatenate([edge_index1[1], edge_index2[1]])
    brow = jnp.stack([x1_batch, x2_batch])[:, None, :]

    w0t = jnp.stack([e1_lin0_W.T, e2_lin0_W.T])
    b0 = jnp.stack([e1_lin0_b, e2_lin0_b])[:, None, :]
    cw = jnp.stack([e1_conv_W, e2_conv_W])
    cb = jnp.stack([e1_conv_b, e2_conv_b])[:, None, :]
    wiht = jnp.stack([e1_gru_Wih.T, e2_gru_Wih.T])
    whht = jnp.stack([e1_gru_Whh.T, e2_gru_Whh.T])
    bih = jnp.stack([e1_gru_bih, e2_gru_bih])[:, None, :]
    bhh = jnp.stack([e1_gru_bhh, e2_gru_bhh])[:, None, :]
    lwiht = jnp.stack([e1_lstm_Wih.T, e2_lstm_Wih.T])
    lwhht = jnp.stack([e1_lstm_Whh.T, e2_lstm_Whh.T])
    lbih = jnp.stack([e1_lstm_bih, e2_lstm_bih])[:, None, :]
    lbhh = jnp.stack([e1_lstm_bhh, e2_lstm_bhh])[:, None, :]

    zeros_mat = jnp.zeros((NPAD, D), jnp.float32)

    src2d = src.reshape(-1, ECHUNK)
    dst2d = dst.reshape(-1, ECHUNK)

    deg = _sc_degree(dst)
    deg = deg.reshape(2, DEG_PAD)[:, :N, None]

    h, dinv, y = _tc_pre(x, w0t, b0, cw, deg)
    for layer in range(3):
        agg = _sc_edge_agg(y.reshape(2 * N, D), src2d, dst2d, zeros_mat)
        want_y = layer < 2
        outs = _tc_gru(y, agg, dinv, cb, h, wiht, whht, bih, bhh, cw, want_y)
        if want_y:
            h, y = outs
        else:
            h = outs[0]

    out = _tc_final(h, brow, lwiht, lwhht, lbih, lbhh,
                    fc1_W.T, fc1_b[None, :], fc2_W.T, fc2_b[None, :])
    return out.reshape(-1)


# speedup vs baseline: 18.7327x; 1.1018x over previous
import functools

import jax
import jax.numpy as jnp
from jax import lax
from jax.experimental import pallas as pl
from jax.experimental.pallas import tpu as pltpu
from jax.experimental.pallas import tpu_sc as plsc

N = 10000
E = 320000
D = 128
B = 16

NUM_TILES = 16
EDGES_PER_TILE = E // NUM_TILES
CHUNK = 80
NCHUNK = EDGES_PER_TILE // CHUNK
NPAD = 10240
ROWS_PER_TILE = NPAD // NUM_TILES
DEG_PAD = 10240

_HIGH = lax.Precision.HIGHEST


def _sc_mesh():
    return plsc.VectorSubcoreMesh(core_axis_name="c", subcore_axis_name="s")


DFIRE = 8


@functools.partial(
    pl.kernel,
    out_type=jax.ShapeDtypeStruct((2 * DEG_PAD,), jnp.float32),
    mesh=_sc_mesh(),
    scratch_types=[
        pltpu.VMEM_SHARED((DEG_PAD,), jnp.float32),
        pltpu.VMEM((EDGES_PER_TILE // 100, 100), jnp.int32),
        pltpu.VMEM((112,), jnp.float32),
        pltpu.VMEM((640,), jnp.float32),
        pltpu.SemaphoreType.DMA,
    ],
)
def _sc_degree(dst_hbm, out_hbm, acc, didx, ones_v, zbuf, sem):
    c = lax.axis_index("c")
    s = lax.axis_index("s")
    nrows = EDGES_PER_TILE // 100
    for j in range(640 // 16):
        zbuf[pl.ds(j * 16, 16)] = jnp.zeros((16,), jnp.float32)
    for j in range(112 // 16):
        ones_v[pl.ds(j * 16, 16)] = jnp.ones((16,), jnp.float32)
    rowbase = pl.multiple_of(c * (E // 100) + s * nrows, 8)
    pltpu.sync_copy(dst_hbm.at[pl.ds(rowbase, nrows)], didx)
    pltpu.sync_copy(zbuf, acc.at[pl.ds(s * 640, 640)])
    plsc.subcore_barrier()

    ones100 = ones_v.at[pl.ds(0, 100)]

    def body(i, carry):
        for k in range(DFIRE):
            pltpu.async_copy(ones100, acc.at[didx.at[i * DFIRE + k]], sem,
                             add=True)
        for k in range(DFIRE):
            pltpu.make_async_copy(ones100, acc.at[didx.at[i * DFIRE + k]],
                                  sem).wait()
        return carry

    lax.fori_loop(0, nrows // DFIRE, body, 0)
    plsc.subcore_barrier()
    obase = pl.multiple_of(c * DEG_PAD + s * 640, 8)
    pltpu.sync_copy(acc.at[pl.ds(s * 640, 640)], out_hbm.at[pl.ds(obase, 640)])


ECHUNK = 100
EROWS = EDGES_PER_TILE // ECHUNK
ROWS_PER_SC = E // ECHUNK
SB = 40


@functools.partial(
    pl.kernel,
    out_type=jax.ShapeDtypeStruct((2, NPAD, D), jnp.float32),
    mesh=_sc_mesh(),
    scratch_types=[
        pltpu.VMEM_SHARED((NPAD, D), jnp.float32),
        pltpu.VMEM((2, SB, ECHUNK), jnp.int32),
        pltpu.VMEM((2, SB, ECHUNK), jnp.int32),
        pltpu.VMEM((ECHUNK, D), jnp.float32),
        pltpu.VMEM((ECHUNK, D), jnp.float32),
        pltpu.SemaphoreType.DMA,
        pltpu.SemaphoreType.DMA,
        pltpu.SemaphoreType.DMA,
    ],
)
def _sc_edge_agg(y_hbm, src_hbm, dst_hbm, zeros_hbm, out_hbm,
                 acc, sidx, didx, rows_a, rows_b, sem_a, sem_b, sem_i):
    c = lax.axis_index("c")
    s = lax.axis_index("s")
    tilebase = c * ROWS_PER_SC + s * EROWS
    pltpu.sync_copy(src_hbm.at[pl.ds(pl.multiple_of(tilebase, 8), SB)],
                    sidx.at[0])
    pltpu.sync_copy(dst_hbm.at[pl.ds(pl.multiple_of(tilebase, 8), SB)],
                    didx.at[0])
    pltpu.sync_copy(zeros_hbm.at[pl.ds(s * ROWS_PER_TILE, ROWS_PER_TILE)],
                    acc.at[pl.ds(s * ROWS_PER_TILE, ROWS_PER_TILE)])
    plsc.subcore_barrier()

    def superblock(ob, carry):
        b = jnp.bitwise_and(ob, 1)
        nb = 1 - b
        sb = sidx.at[b]
        db = didx.at[b]

        @pl.when(ob > 0)
        def _():
            rb = pl.multiple_of(tilebase + ob * SB, 8)
            pltpu.make_async_copy(src_hbm.at[pl.ds(rb, SB)], sidx.at[b],
                                  sem_i).wait()
            pltpu.make_async_copy(dst_hbm.at[pl.ds(rb, SB)], didx.at[b],
                                  sem_i).wait()

        @pl.when(ob < EROWS // SB - 1)
        def _():
            rb = pl.multiple_of(tilebase + (ob + 1) * SB, 8)
            pltpu.async_copy(src_hbm.at[pl.ds(rb, SB)], sidx.at[nb], sem_i)
            pltpu.async_copy(dst_hbm.at[pl.ds(rb, SB)], didx.at[nb], sem_i)

        pltpu.async_copy(y_hbm.at[sb.at[0]], rows_a, sem_a)

        def pair(p, carry2):
            j0 = 2 * p
            pltpu.async_copy(y_hbm.at[sb.at[j0 + 1]], rows_b, sem_b)
            pltpu.make_async_copy(y_hbm.at[sb.at[j0]], rows_a, sem_a).wait()
            pltpu.sync_copy(rows_a, acc.at[db.at[j0]], add=True)

            @pl.when(p < SB // 2 - 1)
            def _():
                pltpu.async_copy(y_hbm.at[sb.at[j0 + 2]], rows_a, sem_a)

            pltpu.make_async_copy(y_hbm.at[sb.at[j0 + 1]], rows_b, sem_b).wait()
            pltpu.sync_copy(rows_b, acc.at[db.at[j0 + 1]], add=True)
            return carry2

        lax.fori_loop(0, SB // 2, pair, 0)
        return carry

    lax.fori_loop(0, EROWS // SB, superblock, 0)
    plsc.subcore_barrier()
    pltpu.sync_copy(acc.at[pl.ds(s * ROWS_PER_TILE, ROWS_PER_TILE)],
                    out_hbm.at[c, pl.ds(s * ROWS_PER_TILE, ROWS_PER_TILE)])


ROW_BLK = 2000
NROW = N // ROW_BLK


def _pre_body(x_ref, w0t_ref, b0_ref, cw_ref, deg_ref, h0_ref, dinv_ref, y_ref):
    x = x_ref[0]
    out0 = jnp.maximum(
        jnp.dot(x, w0t_ref[0], precision=_HIGH) + b0_ref[0], 0.0)
    dinv = lax.rsqrt(1.0 + deg_ref[0])
    h0_ref[0] = out0
    dinv_ref[0] = dinv
    y_ref[0] = dinv * jnp.dot(out0, cw_ref[0], precision=_HIGH)


def _tc_pre(x, w0t, b0, cw, deg):
    eb = lambda e, r: (e, 0, 0)
    rb = lambda e, r: (e, r, 0)
    return pl.pallas_call(
        _pre_body,
        grid=(2, NROW),
        in_specs=[
            pl.BlockSpec((1, ROW_BLK, D), rb),
            pl.BlockSpec((1, D, D), eb),
            pl.BlockSpec((1, 1, D), eb),
            pl.BlockSpec((1, D, D), eb),
            pl.BlockSpec((1, ROW_BLK, 1), rb),
        ],
        out_specs=[
            pl.BlockSpec((1, ROW_BLK, D), rb),
            pl.BlockSpec((1, ROW_BLK, 1), rb),
            pl.BlockSpec((1, ROW_BLK, D), rb),
        ],
        out_shape=[
            jax.ShapeDtypeStruct((2, N, D), jnp.float32),
            jax.ShapeDtypeStruct((2, N, 1), jnp.float32),
            jax.ShapeDtypeStruct((2, N, D), jnp.float32),
        ],
    )(x, w0t, b0, cw, deg)


def _gru_body(want_y, y_ref, agg_ref, dinv_ref, cb_ref, h_ref,
              wiht_ref, whht_ref, bih_ref, bhh_ref, cw_ref, *out_refs):
    dinv = dinv_ref[0]
    m = jnp.maximum(dinv * (y_ref[0] + agg_ref[0]) + cb_ref[0], 0.0)
    gi = jnp.dot(m, wiht_ref[0], precision=_HIGH) + bih_ref[0]
    gh = jnp.dot(h_ref[0], whht_ref[0], precision=_HIGH) + bhh_ref[0]
    r = jax.nn.sigmoid(gi[:, :D] + gh[:, :D])
    z = jax.nn.sigmoid(gi[:, D:2 * D] + gh[:, D:2 * D])
    n = jnp.tanh(gi[:, 2 * D:] + r * gh[:, 2 * D:])
    h_new = (1.0 - z) * n + z * h_ref[0]
    out_refs[0][0] = h_new
    if want_y:
        out_refs[1][0] = dinv * jnp.dot(h_new, cw_ref[0], precision=_HIGH)


def _tc_gru(y, agg, dinv, cb, h, wiht, whht, bih, bhh, cw, want_y):
    eb = lambda e, r: (e, 0, 0)
    rb = lambda e, r: (e, r, 0)
    out_specs = [pl.BlockSpec((1, ROW_BLK, D), rb)]
    out_shape = [jax.ShapeDtypeStruct((2, N, D), jnp.float32)]
    if want_y:
        out_specs.append(pl.BlockSpec((1, ROW_BLK, D), rb))
        out_shape.append(jax.ShapeDtypeStruct((2, N, D), jnp.float32))
    return pl.pallas_call(
        functools.partial(_gru_body, want_y),
        grid=(2, NROW),
        in_specs=[
            pl.BlockSpec((1, ROW_BLK, D), rb),
            pl.BlockSpec((1, ROW_BLK, D), rb),
            pl.BlockSpec((1, ROW_BLK, 1), rb),
            pl.BlockSpec((1, 1, D), eb),
            pl.BlockSpec((1, ROW_BLK, D), rb),
            pl.BlockSpec((1, D, 3 * D), eb),
            pl.BlockSpec((1, D, 3 * D), eb),
            pl.BlockSpec((1, 1, 3 * D), eb),
            pl.BlockSpec((1, 1, 3 * D), eb),
            pl.BlockSpec((1, D, D), eb),
        ],
        out_specs=out_specs,
        out_shape=out_shape,
    )(y, agg, dinv, cb, h, wiht, whht, bih, bhh, cw)


def _final_body(h_ref, brow_ref, wiht_ref, whht_ref, bih_ref, bhh_ref,
                f1t_ref, f1b_ref, f2t_ref, f2b_ref, out_ref):
    q_stars = []
    for e in range(2):
        x = h_ref[e]
        brow = brow_ref[e]
        gid = lax.broadcasted_iota(jnp.int32, (B, N), 0)
        maskT = brow == gid
        hl = jnp.zeros((B, D), jnp.float32)
        cl = jnp.zeros((B, D), jnp.float32)
        q_star = jnp.zeros((B, 2 * D), jnp.float32)
        for _ in range(3):
            gates = (jnp.dot(q_star, wiht_ref[e], precision=_HIGH) + bih_ref[e]
                     + jnp.dot(hl, whht_ref[e], precision=_HIGH) + bhh_ref[e])
            ii = jax.nn.sigmoid(gates[:, :D])
            ff = jax.nn.sigmoid(gates[:, D:2 * D])
            gg = jnp.tanh(gates[:, 2 * D:3 * D])
            oo = jax.nn.sigmoid(gates[:, 3 * D:])
            cl = ff * cl + ii * gg
            hl = oo * jnp.tanh(cl)
            q = hl
            st = lax.dot_general(q, x, (((1,), (1,)), ((), ())),
                                 precision=_HIGH)
            smt = jnp.where(maskT, st, -jnp.inf)
            emax = jnp.max(smt, axis=1, keepdims=True)
            emax = jnp.where(jnp.isfinite(emax), emax, 0.0)
            pt = jnp.exp(smt - emax)
            denom = jnp.sum(pt, axis=1, keepdims=True)
            at = pt / (denom + 1e-16)
            r = jnp.dot(at, x, precision=_HIGH)
            q_star = jnp.concatenate([q, r], axis=1)
        q_stars.append(q_star)
    cat = jnp.concatenate(q_stars, axis=1)
    hfc = jnp.maximum(jnp.dot(cat, f1t_ref[...], precision=_HIGH)
                      + f1b_ref[...], 0.0)
    out_ref[...] = jnp.dot(hfc, f2t_ref[...], precision=_HIGH) + f2b_ref[...]


def _tc_final(h, brow, wiht, whht, bih, bhh, f1t, f1b, f2t, f2b):
    return pl.pallas_call(
        _final_body,
        out_shape=jax.ShapeDtypeStruct((B, 1), jnp.float32),
    )(h, brow, wiht, whht, bih, bhh, f1t, f1b, f2t, f2b)


def kernel(x1, x2, edge_index1, edge_index2, x1_batch, x2_batch,
           e1_lin0_W, e1_lin0_b, e1_conv_W, e1_conv_b,
           e1_gru_Wih, e1_gru_Whh, e1_gru_bih, e1_gru_bhh,
           e1_lstm_Wih, e1_lstm_Whh, e1_lstm_bih, e1_lstm_bhh,
           e2_lin0_W, e2_lin0_b, e2_conv_W, e2_conv_b,
           e2_gru_Wih, e2_gru_Whh, e2_gru_bih, e2_gru_bhh,
           e2_lstm_Wih, e2_lstm_Whh, e2_lstm_bih, e2_lstm_bhh,
           fc1_W, fc1_b, fc2_W, fc2_b):
    x = jnp.stack([x1, x2])
    src = jnp.concatenate([edge_index1[0], edge_index2[0] + N])
    dst = jnp.concatenate([edge_index1[1], edge_index2[1]])
    brow = jnp.stack([x1_batch, x2_batch])[:, None, :]

    w0t = jnp.stack([e1_lin0_W.T, e2_lin0_W.T])
    b0 = jnp.stack([e1_lin0_b, e2_lin0_b])[:, None, :]
    cw = jnp.stack([e1_conv_W, e2_conv_W])
    cb = jnp.stack([e1_conv_b, e2_conv_b])[:, None, :]
    wiht = jnp.stack([e1_gru_Wih.T, e2_gru_Wih.T])
    whht = jnp.stack([e1_gru_Whh.T, e2_gru_Whh.T])
    bih = jnp.stack([e1_gru_bih, e2_gru_bih])[:, None, :]
    bhh = jnp.stack([e1_gru_bhh, e2_gru_bhh])[:, None, :]
    lwiht = jnp.stack([e1_lstm_Wih.T, e2_lstm_Wih.T])
    lwhht = jnp.stack([e1_lstm_Whh.T, e2_lstm_Whh.T])
    lbih = jnp.stack([e1_lstm_bih, e2_lstm_bih])[:, None, :]
    lbhh = jnp.stack([e1_lstm_bhh, e2_lstm_bhh])[:, None, :]

    zeros_mat = jnp.zeros((NPAD, D), jnp.float32)

    src2d = src.reshape(-1, ECHUNK)
    dst2d = dst.reshape(-1, ECHUNK)

    deg = _sc_degree(dst2d)
    deg = deg.reshape(2, DEG_PAD)[:, :N, None]

    h, dinv, y = _tc_pre(x, w0t, b0, cw, deg)
    for layer in range(3):
        agg = _sc_edge_agg(y.reshape(2 * N, D), src2d, dst2d, zeros_mat)
        want_y = layer < 2
        outs = _tc_gru(y, agg, dinv, cb, h, wiht, whht, bih, bhh, cw, want_y)
        if want_y:
            h, y = outs
        else:
            h = outs[0]

    out = _tc_final(h, brow, lwiht, lwhht, lbih, lbhh,
                    fc1_W.T, fc1_b[None, :], fc2_W.T, fc2_b[None, :])
    return out.reshape(-1)


# speedup vs baseline: 23.1029x; 1.2333x over previous
import functools

import jax
import jax.numpy as jnp
from jax import lax
from jax.experimental import pallas as pl
from jax.experimental.pallas import tpu as pltpu
from jax.experimental.pallas import tpu_sc as plsc

N = 10000
E = 320000
D = 128
B = 16

NUM_TILES = 16
EDGES_PER_TILE = E // NUM_TILES
CHUNK = 80
NCHUNK = EDGES_PER_TILE // CHUNK
NPAD = 10240
ROWS_PER_TILE = NPAD // NUM_TILES
DEG_PAD = 10240

_HIGH = lax.Precision.DEFAULT


def _sc_mesh():
    return plsc.VectorSubcoreMesh(core_axis_name="c", subcore_axis_name="s")


DFIRE = 8


@functools.partial(
    pl.kernel,
    out_type=jax.ShapeDtypeStruct((2 * DEG_PAD,), jnp.float32),
    mesh=_sc_mesh(),
    scratch_types=[
        pltpu.VMEM_SHARED((DEG_PAD,), jnp.float32),
        pltpu.VMEM((EDGES_PER_TILE // 100, 100), jnp.int32),
        pltpu.VMEM((112,), jnp.float32),
        pltpu.VMEM((640,), jnp.float32),
        pltpu.SemaphoreType.DMA,
    ],
)
def _sc_degree(dst_hbm, out_hbm, acc, didx, ones_v, zbuf, sem):
    c = lax.axis_index("c")
    s = lax.axis_index("s")
    nrows = EDGES_PER_TILE // 100
    for j in range(640 // 16):
        zbuf[pl.ds(j * 16, 16)] = jnp.zeros((16,), jnp.float32)
    for j in range(112 // 16):
        ones_v[pl.ds(j * 16, 16)] = jnp.ones((16,), jnp.float32)
    rowbase = pl.multiple_of(c * (E // 100) + s * nrows, 8)
    pltpu.sync_copy(dst_hbm.at[pl.ds(rowbase, nrows)], didx)
    pltpu.sync_copy(zbuf, acc.at[pl.ds(s * 640, 640)])
    plsc.subcore_barrier()

    ones100 = ones_v.at[pl.ds(0, 100)]

    def body(i, carry):
        for k in range(DFIRE):
            pltpu.async_copy(ones100, acc.at[didx.at[i * DFIRE + k]], sem,
                             add=True)
        for k in range(DFIRE):
            pltpu.make_async_copy(ones100, acc.at[didx.at[i * DFIRE + k]],
                                  sem).wait()
        return carry

    lax.fori_loop(0, nrows // DFIRE, body, 0)
    plsc.subcore_barrier()
    obase = pl.multiple_of(c * DEG_PAD + s * 640, 8)
    pltpu.sync_copy(acc.at[pl.ds(s * 640, 640)], out_hbm.at[pl.ds(obase, 640)])


ECHUNK = 100
EROWS = EDGES_PER_TILE // ECHUNK
ROWS_PER_SC = E // ECHUNK
SB = 40


@functools.partial(
    pl.kernel,
    out_type=jax.ShapeDtypeStruct((2, NPAD, D), jnp.float32),
    mesh=_sc_mesh(),
    scratch_types=[
        pltpu.VMEM_SHARED((NPAD, D), jnp.float32),
        pltpu.VMEM((2, SB, ECHUNK), jnp.int32),
        pltpu.VMEM((2, SB, ECHUNK), jnp.int32),
        pltpu.VMEM((ECHUNK, D), jnp.float32),
        pltpu.VMEM((ECHUNK, D), jnp.float32),
        pltpu.SemaphoreType.DMA,
        pltpu.SemaphoreType.DMA,
        pltpu.SemaphoreType.DMA,
    ],
)
def _sc_edge_agg(y_hbm, src_hbm, dst_hbm, zeros_hbm, out_hbm,
                 acc, sidx, didx, rows_a, rows_b, sem_a, sem_b, sem_i):
    c = lax.axis_index("c")
    s = lax.axis_index("s")
    tilebase = c * ROWS_PER_SC + s * EROWS
    pltpu.sync_copy(src_hbm.at[pl.ds(pl.multiple_of(tilebase, 8), SB)],
                    sidx.at[0])
    pltpu.sync_copy(dst_hbm.at[pl.ds(pl.multiple_of(tilebase, 8), SB)],
                    didx.at[0])
    pltpu.sync_copy(zeros_hbm.at[pl.ds(s * ROWS_PER_TILE, ROWS_PER_TILE)],
                    acc.at[pl.ds(s * ROWS_PER_TILE, ROWS_PER_TILE)])
    plsc.subcore_barrier()

    def superblock(ob, carry):
        b = jnp.bitwise_and(ob, 1)
        nb = 1 - b
        sb = sidx.at[b]
        db = didx.at[b]

        @pl.when(ob > 0)
        def _():
            rb = pl.multiple_of(tilebase + ob * SB, 8)
            pltpu.make_async_copy(src_hbm.at[pl.ds(rb, SB)], sidx.at[b],
                                  sem_i).wait()
            pltpu.make_async_copy(dst_hbm.at[pl.ds(rb, SB)], didx.at[b],
                                  sem_i).wait()

        @pl.when(ob < EROWS // SB - 1)
        def _():
            rb = pl.multiple_of(tilebase + (ob + 1) * SB, 8)
            pltpu.async_copy(src_hbm.at[pl.ds(rb, SB)], sidx.at[nb], sem_i)
            pltpu.async_copy(dst_hbm.at[pl.ds(rb, SB)], didx.at[nb], sem_i)

        pltpu.async_copy(y_hbm.at[sb.at[0]], rows_a, sem_a)

        def pair(p, carry2):
            j0 = 2 * p
            pltpu.async_copy(y_hbm.at[sb.at[j0 + 1]], rows_b, sem_b)
            pltpu.make_async_copy(y_hbm.at[sb.at[j0]], rows_a, sem_a).wait()
            pltpu.sync_copy(rows_a, acc.at[db.at[j0]], add=True)

            @pl.when(p < SB // 2 - 1)
            def _():
                pltpu.async_copy(y_hbm.at[sb.at[j0 + 2]], rows_a, sem_a)

            pltpu.make_async_copy(y_hbm.at[sb.at[j0 + 1]], rows_b, sem_b).wait()
            pltpu.sync_copy(rows_b, acc.at[db.at[j0 + 1]], add=True)
            return carry2

        lax.fori_loop(0, SB // 2, pair, 0)
        return carry

    lax.fori_loop(0, EROWS // SB, superblock, 0)
    plsc.subcore_barrier()
    pltpu.sync_copy(acc.at[pl.ds(s * ROWS_PER_TILE, ROWS_PER_TILE)],
                    out_hbm.at[c, pl.ds(s * ROWS_PER_TILE, ROWS_PER_TILE)])


ROW_BLK = 2000
NROW = N // ROW_BLK


def _pre_body(x_ref, w0t_ref, b0_ref, cw_ref, deg_ref, h0_ref, dinv_ref, y_ref):
    x = x_ref[0]
    out0 = jnp.maximum(
        jnp.dot(x, w0t_ref[0], precision=_HIGH) + b0_ref[0], 0.0)
    dinv = lax.rsqrt(1.0 + deg_ref[0])
    h0_ref[0] = out0
    dinv_ref[0] = dinv
    y_ref[0] = dinv * jnp.dot(out0, cw_ref[0], precision=_HIGH)


def _tc_pre(x, w0t, b0, cw, deg):
    eb = lambda e, r: (e, 0, 0)
    rb = lambda e, r: (e, r, 0)
    return pl.pallas_call(
        _pre_body,
        grid=(2, NROW),
        in_specs=[
            pl.BlockSpec((1, ROW_BLK, D), rb),
            pl.BlockSpec((1, D, D), eb),
            pl.BlockSpec((1, 1, D), eb),
            pl.BlockSpec((1, D, D), eb),
            pl.BlockSpec((1, ROW_BLK, 1), rb),
        ],
        out_specs=[
            pl.BlockSpec((1, ROW_BLK, D), rb),
            pl.BlockSpec((1, ROW_BLK, 1), rb),
            pl.BlockSpec((1, ROW_BLK, D), rb),
        ],
        out_shape=[
            jax.ShapeDtypeStruct((2, N, D), jnp.float32),
            jax.ShapeDtypeStruct((2, N, 1), jnp.float32),
            jax.ShapeDtypeStruct((2, N, D), jnp.float32),
        ],
    )(x, w0t, b0, cw, deg)


def _gru_body(want_y, y_ref, agg_ref, dinv_ref, cb_ref, h_ref,
              wiht_ref, whht_ref, bih_ref, bhh_ref, cw_ref, *out_refs):
    dinv = dinv_ref[0]
    m = jnp.maximum(dinv * (y_ref[0] + agg_ref[0]) + cb_ref[0], 0.0)
    gi = jnp.dot(m, wiht_ref[0], precision=_HIGH) + bih_ref[0]
    gh = jnp.dot(h_ref[0], whht_ref[0], precision=_HIGH) + bhh_ref[0]
    r = jax.nn.sigmoid(gi[:, :D] + gh[:, :D])
    z = jax.nn.sigmoid(gi[:, D:2 * D] + gh[:, D:2 * D])
    n = jnp.tanh(gi[:, 2 * D:] + r * gh[:, 2 * D:])
    h_new = (1.0 - z) * n + z * h_ref[0]
    out_refs[0][0] = h_new
    if want_y:
        out_refs[1][0] = dinv * jnp.dot(h_new, cw_ref[0], precision=_HIGH)


def _tc_gru(y, agg, dinv, cb, h, wiht, whht, bih, bhh, cw, want_y):
    eb = lambda e, r: (e, 0, 0)
    rb = lambda e, r: (e, r, 0)
    out_specs = [pl.BlockSpec((1, ROW_BLK, D), rb)]
    out_shape = [jax.ShapeDtypeStruct((2, N, D), jnp.float32)]
    if want_y:
        out_specs.append(pl.BlockSpec((1, ROW_BLK, D), rb))
        out_shape.append(jax.ShapeDtypeStruct((2, N, D), jnp.float32))
    return pl.pallas_call(
        functools.partial(_gru_body, want_y),
        grid=(2, NROW),
        in_specs=[
            pl.BlockSpec((1, ROW_BLK, D), rb),
            pl.BlockSpec((1, ROW_BLK, D), rb),
            pl.BlockSpec((1, ROW_BLK, 1), rb),
            pl.BlockSpec((1, 1, D), eb),
            pl.BlockSpec((1, ROW_BLK, D), rb),
            pl.BlockSpec((1, D, 3 * D), eb),
            pl.BlockSpec((1, D, 3 * D), eb),
            pl.BlockSpec((1, 1, 3 * D), eb),
            pl.BlockSpec((1, 1, 3 * D), eb),
            pl.BlockSpec((1, D, D), eb),
        ],
        out_specs=out_specs,
        out_shape=out_shape,
    )(y, agg, dinv, cb, h, wiht, whht, bih, bhh, cw)


def _final_body(h_ref, brow_ref, wiht_ref, whht_ref, bih_ref, bhh_ref,
                f1t_ref, f1b_ref, f2t_ref, f2b_ref, out_ref):
    q_stars = []
    for e in range(2):
        x = h_ref[e]
        brow = brow_ref[e]
        gid = lax.broadcasted_iota(jnp.int32, (B, N), 0)
        maskT = brow == gid
        hl = jnp.zeros((B, D), jnp.float32)
        cl = jnp.zeros((B, D), jnp.float32)
        q_star = jnp.zeros((B, 2 * D), jnp.float32)
        for _ in range(3):
            gates = (jnp.dot(q_star, wiht_ref[e], precision=_HIGH) + bih_ref[e]
                     + jnp.dot(hl, whht_ref[e], precision=_HIGH) + bhh_ref[e])
            ii = jax.nn.sigmoid(gates[:, :D])
            ff = jax.nn.sigmoid(gates[:, D:2 * D])
            gg = jnp.tanh(gates[:, 2 * D:3 * D])
            oo = jax.nn.sigmoid(gates[:, 3 * D:])
            cl = ff * cl + ii * gg
            hl = oo * jnp.tanh(cl)
            q = hl
            st = lax.dot_general(q, x, (((1,), (1,)), ((), ())),
                                 precision=_HIGH)
            smt = jnp.where(maskT, st, -jnp.inf)
            emax = jnp.max(smt, axis=1, keepdims=True)
            emax = jnp.where(jnp.isfinite(emax), emax, 0.0)
            pt = jnp.exp(smt - emax)
            denom = jnp.sum(pt, axis=1, keepdims=True)
            at = pt / (denom + 1e-16)
            r = jnp.dot(at, x, precision=_HIGH)
            q_star = jnp.concatenate([q, r], axis=1)
        q_stars.append(q_star)
    cat = jnp.concatenate(q_stars, axis=1)
    hfc = jnp.maximum(jnp.dot(cat, f1t_ref[...], precision=_HIGH)
                      + f1b_ref[...], 0.0)
    out_ref[...] = jnp.dot(hfc, f2t_ref[...], precision=_HIGH) + f2b_ref[...]


def _tc_final(h, brow, wiht, whht, bih, bhh, f1t, f1b, f2t, f2b):
    return pl.pallas_call(
        _final_body,
        out_shape=jax.ShapeDtypeStruct((B, 1), jnp.float32),
    )(h, brow, wiht, whht, bih, bhh, f1t, f1b, f2t, f2b)


def kernel(x1, x2, edge_index1, edge_index2, x1_batch, x2_batch,
           e1_lin0_W, e1_lin0_b, e1_conv_W, e1_conv_b,
           e1_gru_Wih, e1_gru_Whh, e1_gru_bih, e1_gru_bhh,
           e1_lstm_Wih, e1_lstm_Whh, e1_lstm_bih, e1_lstm_bhh,
           e2_lin0_W, e2_lin0_b, e2_conv_W, e2_conv_b,
           e2_gru_Wih, e2_gru_Whh, e2_gru_bih, e2_gru_bhh,
           e2_lstm_Wih, e2_lstm_Whh, e2_lstm_bih, e2_lstm_bhh,
           fc1_W, fc1_b, fc2_W, fc2_b):
    x = jnp.stack([x1, x2])
    src = jnp.concatenate([edge_index1[0], edge_index2[0] + N])
    dst = jnp.concatenate([edge_index1[1], edge_index2[1]])
    brow = jnp.stack([x1_batch, x2_batch])[:, None, :]

    w0t = jnp.stack([e1_lin0_W.T, e2_lin0_W.T])
    b0 = jnp.stack([e1_lin0_b, e2_lin0_b])[:, None, :]
    cw = jnp.stack([e1_conv_W, e2_conv_W])
    cb = jnp.stack([e1_conv_b, e2_conv_b])[:, None, :]
    wiht = jnp.stack([e1_gru_Wih.T, e2_gru_Wih.T])
    whht = jnp.stack([e1_gru_Whh.T, e2_gru_Whh.T])
    bih = jnp.stack([e1_gru_bih, e2_gru_bih])[:, None, :]
    bhh = jnp.stack([e1_gru_bhh, e2_gru_bhh])[:, None, :]
    lwiht = jnp.stack([e1_lstm_Wih.T, e2_lstm_Wih.T])
    lwhht = jnp.stack([e1_lstm_Whh.T, e2_lstm_Whh.T])
    lbih = jnp.stack([e1_lstm_bih, e2_lstm_bih])[:, None, :]
    lbhh = jnp.stack([e1_lstm_bhh, e2_lstm_bhh])[:, None, :]

    zeros_mat = jnp.zeros((NPAD, D), jnp.float32)

    src2d = src.reshape(-1, ECHUNK)
    dst2d = dst.reshape(-1, ECHUNK)

    deg = _sc_degree(dst2d)
    deg = deg.reshape(2, DEG_PAD)[:, :N, None]

    h, dinv, y = _tc_pre(x, w0t, b0, cw, deg)
    for layer in range(3):
        agg = _sc_edge_agg(y.reshape(2 * N, D), src2d, dst2d, zeros_mat)
        want_y = layer < 2
        outs = _tc_gru(y, agg, dinv, cb, h, wiht, whht, bih, bhh, cw, want_y)
        if want_y:
            h, y = outs
        else:
            h = outs[0]

    out = _tc_final(h, brow, lwiht, lwhht, lbih, lbhh,
                    fc1_W.T, fc1_b[None, :], fc2_W.T, fc2_b[None, :])
    return out.reshape(-1)


# speedup vs baseline: 23.3888x; 1.0124x over previous
import functools

import jax
import jax.numpy as jnp
from jax import lax
from jax.experimental import pallas as pl
from jax.experimental.pallas import tpu as pltpu
from jax.experimental.pallas import tpu_sc as plsc

N = 10000
E = 320000
D = 128
B = 16

NUM_TILES = 16
EDGES_PER_TILE = E // NUM_TILES
CHUNK = 80
NCHUNK = EDGES_PER_TILE // CHUNK
NPAD = 10240
ROWS_PER_TILE = NPAD // NUM_TILES
DEG_PAD = 10240
ECHUNK = 125

_HIGH = lax.Precision.DEFAULT


def _sc_mesh():
    return plsc.VectorSubcoreMesh(core_axis_name="c", subcore_axis_name="s")


DFIRE = 8


@functools.partial(
    pl.kernel,
    out_type=jax.ShapeDtypeStruct((2 * DEG_PAD,), jnp.float32),
    mesh=_sc_mesh(),
    scratch_types=[
        pltpu.VMEM_SHARED((DEG_PAD,), jnp.float32),
        pltpu.VMEM((EDGES_PER_TILE // ECHUNK, ECHUNK), jnp.int32),
        pltpu.VMEM((128,), jnp.float32),
        pltpu.VMEM((640,), jnp.float32),
        pltpu.SemaphoreType.DMA,
    ],
)
def _sc_degree(dst_hbm, out_hbm, acc, didx, ones_v, zbuf, sem):
    c = lax.axis_index("c")
    s = lax.axis_index("s")
    nrows = EDGES_PER_TILE // ECHUNK
    for j in range(640 // 16):
        zbuf[pl.ds(j * 16, 16)] = jnp.zeros((16,), jnp.float32)
    for j in range(128 // 16):
        ones_v[pl.ds(j * 16, 16)] = jnp.ones((16,), jnp.float32)
    rowbase = pl.multiple_of(c * (E // ECHUNK) + s * nrows, 8)
    pltpu.sync_copy(dst_hbm.at[pl.ds(rowbase, nrows)], didx)
    pltpu.sync_copy(zbuf, acc.at[pl.ds(s * 640, 640)])
    plsc.subcore_barrier()

    ones100 = ones_v.at[pl.ds(0, ECHUNK)]

    def body(i, carry):
        for k in range(DFIRE):
            pltpu.async_copy(ones100, acc.at[didx.at[i * DFIRE + k]], sem,
                             add=True)
        for k in range(DFIRE):
            pltpu.make_async_copy(ones100, acc.at[didx.at[i * DFIRE + k]],
                                  sem).wait()
        return carry

    lax.fori_loop(0, nrows // DFIRE, body, 0)
    plsc.subcore_barrier()
    obase = pl.multiple_of(c * DEG_PAD + s * 640, 8)
    pltpu.sync_copy(acc.at[pl.ds(s * 640, 640)], out_hbm.at[pl.ds(obase, 640)])


EROWS = EDGES_PER_TILE // ECHUNK
ROWS_PER_SC = E // ECHUNK
SB = 16


@functools.partial(
    pl.kernel,
    out_type=jax.ShapeDtypeStruct((2, NPAD, D), jnp.float32),
    mesh=_sc_mesh(),
    scratch_types=[
        pltpu.VMEM_SHARED((NPAD, D), jnp.float32),
        pltpu.VMEM((2, SB, ECHUNK), jnp.int32),
        pltpu.VMEM((2, SB, ECHUNK), jnp.int32),
        pltpu.VMEM((ECHUNK, D), jnp.float32),
        pltpu.VMEM((ECHUNK, D), jnp.float32),
        pltpu.SemaphoreType.DMA,
        pltpu.SemaphoreType.DMA,
        pltpu.SemaphoreType.DMA,
    ],
)
def _sc_edge_agg(y_hbm, src_hbm, dst_hbm, zeros_hbm, out_hbm,
                 acc, sidx, didx, rows_a, rows_b, sem_a, sem_b, sem_i):
    c = lax.axis_index("c")
    s = lax.axis_index("s")
    tilebase = c * ROWS_PER_SC + s * EROWS
    pltpu.sync_copy(src_hbm.at[pl.ds(pl.multiple_of(tilebase, 8), SB)],
                    sidx.at[0])
    pltpu.sync_copy(dst_hbm.at[pl.ds(pl.multiple_of(tilebase, 8), SB)],
                    didx.at[0])
    pltpu.sync_copy(zeros_hbm.at[pl.ds(s * ROWS_PER_TILE, ROWS_PER_TILE)],
                    acc.at[pl.ds(s * ROWS_PER_TILE, ROWS_PER_TILE)])
    plsc.subcore_barrier()

    def superblock(ob, carry):
        b = jnp.bitwise_and(ob, 1)
        nb = 1 - b
        sb = sidx.at[b]
        db = didx.at[b]

        @pl.when(ob > 0)
        def _():
            rb = pl.multiple_of(tilebase + ob * SB, 8)
            pltpu.make_async_copy(src_hbm.at[pl.ds(rb, SB)], sidx.at[b],
                                  sem_i).wait()
            pltpu.make_async_copy(dst_hbm.at[pl.ds(rb, SB)], didx.at[b],
                                  sem_i).wait()

        @pl.when(ob < EROWS // SB - 1)
        def _():
            rb = pl.multiple_of(tilebase + (ob + 1) * SB, 8)
            pltpu.async_copy(src_hbm.at[pl.ds(rb, SB)], sidx.at[nb], sem_i)
            pltpu.async_copy(dst_hbm.at[pl.ds(rb, SB)], didx.at[nb], sem_i)

        pltpu.async_copy(y_hbm.at[sb.at[0]], rows_a, sem_a)

        def pair(p, carry2):
            j0 = 2 * p
            pltpu.async_copy(y_hbm.at[sb.at[j0 + 1]], rows_b, sem_b)
            pltpu.make_async_copy(y_hbm.at[sb.at[j0]], rows_a, sem_a).wait()
            pltpu.sync_copy(rows_a, acc.at[db.at[j0]], add=True)

            @pl.when(p < SB // 2 - 1)
            def _():
                pltpu.async_copy(y_hbm.at[sb.at[j0 + 2]], rows_a, sem_a)

            pltpu.make_async_copy(y_hbm.at[sb.at[j0 + 1]], rows_b, sem_b).wait()
            pltpu.sync_copy(rows_b, acc.at[db.at[j0 + 1]], add=True)
            return carry2

        lax.fori_loop(0, SB // 2, pair, 0)
        return carry

    lax.fori_loop(0, EROWS // SB, superblock, 0)
    plsc.subcore_barrier()
    pltpu.sync_copy(acc.at[pl.ds(s * ROWS_PER_TILE, ROWS_PER_TILE)],
                    out_hbm.at[c, pl.ds(s * ROWS_PER_TILE, ROWS_PER_TILE)])


ROW_BLK = 2000
NROW = N // ROW_BLK


def _pre_body(x_ref, w0t_ref, b0_ref, cw_ref, deg_ref, h0_ref, dinv_ref, y_ref):
    x = x_ref[0]
    out0 = jnp.maximum(
        jnp.dot(x, w0t_ref[0], precision=_HIGH) + b0_ref[0], 0.0)
    dinv = lax.rsqrt(1.0 + deg_ref[0])
    h0_ref[0] = out0
    dinv_ref[0] = dinv
    y_ref[0] = dinv * jnp.dot(out0, cw_ref[0], precision=_HIGH)


def _tc_pre(x, w0t, b0, cw, deg):
    eb = lambda e, r: (e, 0, 0)
    rb = lambda e, r: (e, r, 0)
    return pl.pallas_call(
        _pre_body,
        grid=(2, NROW),
        in_specs=[
            pl.BlockSpec((1, ROW_BLK, D), rb),
            pl.BlockSpec((1, D, D), eb),
            pl.BlockSpec((1, 1, D), eb),
            pl.BlockSpec((1, D, D), eb),
            pl.BlockSpec((1, ROW_BLK, 1), rb),
        ],
        out_specs=[
            pl.BlockSpec((1, ROW_BLK, D), rb),
            pl.BlockSpec((1, ROW_BLK, 1), rb),
            pl.BlockSpec((1, ROW_BLK, D), rb),
        ],
        out_shape=[
            jax.ShapeDtypeStruct((2, N, D), jnp.float32),
            jax.ShapeDtypeStruct((2, N, 1), jnp.float32),
            jax.ShapeDtypeStruct((2, N, D), jnp.float32),
        ],
    )(x, w0t, b0, cw, deg)


def _gru_body(want_y, y_ref, agg_ref, dinv_ref, cb_ref, h_ref,
              wiht_ref, whht_ref, bih_ref, bhh_ref, cw_ref, *out_refs):
    dinv = dinv_ref[0]
    m = jnp.maximum(dinv * (y_ref[0] + agg_ref[0]) + cb_ref[0], 0.0)
    gi = jnp.dot(m, wiht_ref[0], precision=_HIGH) + bih_ref[0]
    gh = jnp.dot(h_ref[0], whht_ref[0], precision=_HIGH) + bhh_ref[0]
    r = jax.nn.sigmoid(gi[:, :D] + gh[:, :D])
    z = jax.nn.sigmoid(gi[:, D:2 * D] + gh[:, D:2 * D])
    n = jnp.tanh(gi[:, 2 * D:] + r * gh[:, 2 * D:])
    h_new = (1.0 - z) * n + z * h_ref[0]
    out_refs[0][0] = h_new
    if want_y:
        out_refs[1][0] = dinv * jnp.dot(h_new, cw_ref[0], precision=_HIGH)


def _tc_gru(y, agg, dinv, cb, h, wiht, whht, bih, bhh, cw, want_y):
    eb = lambda e, r: (e, 0, 0)
    rb = lambda e, r: (e, r, 0)
    out_specs = [pl.BlockSpec((1, ROW_BLK, D), rb)]
    out_shape = [jax.ShapeDtypeStruct((2, N, D), jnp.float32)]
    if want_y:
        out_specs.append(pl.BlockSpec((1, ROW_BLK, D), rb))
        out_shape.append(jax.ShapeDtypeStruct((2, N, D), jnp.float32))
    return pl.pallas_call(
        functools.partial(_gru_body, want_y),
        grid=(2, NROW),
        in_specs=[
            pl.BlockSpec((1, ROW_BLK, D), rb),
            pl.BlockSpec((1, ROW_BLK, D), rb),
            pl.BlockSpec((1, ROW_BLK, 1), rb),
            pl.BlockSpec((1, 1, D), eb),
            pl.BlockSpec((1, ROW_BLK, D), rb),
            pl.BlockSpec((1, D, 3 * D), eb),
            pl.BlockSpec((1, D, 3 * D), eb),
            pl.BlockSpec((1, 1, 3 * D), eb),
            pl.BlockSpec((1, 1, 3 * D), eb),
            pl.BlockSpec((1, D, D), eb),
        ],
        out_specs=out_specs,
        out_shape=out_shape,
    )(y, agg, dinv, cb, h, wiht, whht, bih, bhh, cw)


def _final_body(h_ref, brow_ref, wiht_ref, whht_ref, bih_ref, bhh_ref,
                f1t_ref, f1b_ref, f2t_ref, f2b_ref, out_ref):
    q_stars = []
    for e in range(2):
        x = h_ref[e]
        brow = brow_ref[e]
        gid = lax.broadcasted_iota(jnp.int32, (B, N), 0)
        maskT = brow == gid
        hl = jnp.zeros((B, D), jnp.float32)
        cl = jnp.zeros((B, D), jnp.float32)
        q_star = jnp.zeros((B, 2 * D), jnp.float32)
        for _ in range(3):
            gates = (jnp.dot(q_star, wiht_ref[e], precision=_HIGH) + bih_ref[e]
                     + jnp.dot(hl, whht_ref[e], precision=_HIGH) + bhh_ref[e])
            ii = jax.nn.sigmoid(gates[:, :D])
            ff = jax.nn.sigmoid(gates[:, D:2 * D])
            gg = jnp.tanh(gates[:, 2 * D:3 * D])
            oo = jax.nn.sigmoid(gates[:, 3 * D:])
            cl = ff * cl + ii * gg
            hl = oo * jnp.tanh(cl)
            q = hl
            st = lax.dot_general(q, x, (((1,), (1,)), ((), ())),
                                 precision=_HIGH)
            smt = jnp.where(maskT, st, -jnp.inf)
            emax = jnp.max(smt, axis=1, keepdims=True)
            emax = jnp.where(jnp.isfinite(emax), emax, 0.0)
            pt = jnp.exp(smt - emax)
            denom = jnp.sum(pt, axis=1, keepdims=True)
            at = pt / (denom + 1e-16)
            r = jnp.dot(at, x, precision=_HIGH)
            q_star = jnp.concatenate([q, r], axis=1)
        q_stars.append(q_star)
    cat = jnp.concatenate(q_stars, axis=1)
    hfc = jnp.maximum(jnp.dot(cat, f1t_ref[...], precision=_HIGH)
                      + f1b_ref[...], 0.0)
    out_ref[...] = jnp.dot(hfc, f2t_ref[...], precision=_HIGH) + f2b_ref[...]


def _tc_final(h, brow, wiht, whht, bih, bhh, f1t, f1b, f2t, f2b):
    return pl.pallas_call(
        _final_body,
        out_shape=jax.ShapeDtypeStruct((B, 1), jnp.float32),
    )(h, brow, wiht, whht, bih, bhh, f1t, f1b, f2t, f2b)


def kernel(x1, x2, edge_index1, edge_index2, x1_batch, x2_batch,
           e1_lin0_W, e1_lin0_b, e1_conv_W, e1_conv_b,
           e1_gru_Wih, e1_gru_Whh, e1_gru_bih, e1_gru_bhh,
           e1_lstm_Wih, e1_lstm_Whh, e1_lstm_bih, e1_lstm_bhh,
           e2_lin0_W, e2_lin0_b, e2_conv_W, e2_conv_b,
           e2_gru_Wih, e2_gru_Whh, e2_gru_bih, e2_gru_bhh,
           e2_lstm_Wih, e2_lstm_Whh, e2_lstm_bih, e2_lstm_bhh,
           fc1_W, fc1_b, fc2_W, fc2_b):
    x = jnp.stack([x1, x2])
    src = jnp.concatenate([edge_index1[0], edge_index2[0] + N])
    dst = jnp.concatenate([edge_index1[1], edge_index2[1]])
    brow = jnp.stack([x1_batch, x2_batch])[:, None, :]

    w0t = jnp.stack([e1_lin0_W.T, e2_lin0_W.T])
    b0 = jnp.stack([e1_lin0_b, e2_lin0_b])[:, None, :]
    cw = jnp.stack([e1_conv_W, e2_conv_W])
    cb = jnp.stack([e1_conv_b, e2_conv_b])[:, None, :]
    wiht = jnp.stack([e1_gru_Wih.T, e2_gru_Wih.T])
    whht = jnp.stack([e1_gru_Whh.T, e2_gru_Whh.T])
    bih = jnp.stack([e1_gru_bih, e2_gru_bih])[:, None, :]
    bhh = jnp.stack([e1_gru_bhh, e2_gru_bhh])[:, None, :]
    lwiht = jnp.stack([e1_lstm_Wih.T, e2_lstm_Wih.T])
    lwhht = jnp.stack([e1_lstm_Whh.T, e2_lstm_Whh.T])
    lbih = jnp.stack([e1_lstm_bih, e2_lstm_bih])[:, None, :]
    lbhh = jnp.stack([e1_lstm_bhh, e2_lstm_bhh])[:, None, :]

    zeros_mat = jnp.zeros((NPAD, D), jnp.float32)

    src2d = src.reshape(-1, ECHUNK)
    dst2d = dst.reshape(-1, ECHUNK)

    deg = _sc_degree(dst2d)
    deg = deg.reshape(2, DEG_PAD)[:, :N, None]

    h, dinv, y = _tc_pre(x, w0t, b0, cw, deg)
    for layer in range(3):
        agg = _sc_edge_agg(y.reshape(2 * N, D), src2d, dst2d, zeros_mat)
        want_y = layer < 2
        outs = _tc_gru(y, agg, dinv, cb, h, wiht, whht, bih, bhh, cw, want_y)
        if want_y:
            h, y = outs
        else:
            h = outs[0]

    out = _tc_final(h, brow, lwiht, lwhht, lbih, lbhh,
                    fc1_W.T, fc1_b[None, :], fc2_W.T, fc2_b[None, :])
    return out.reshape(-1)
